# baseline (speedup 1.0000x reference)
import functools

import jax
import jax.numpy as jnp
from jax import lax
from jax.experimental import pallas as pl
from jax.experimental.pallas import tpu as pltpu
from jax.experimental.pallas import tpu_sc as plsc

NC = 2
NS = 16
NW = NC * NS

BATCH = 64
NSPLIT = 1
BSUB = BATCH // NSPLIT
SEQ = 8192
DIM = 64
PAD = 128
CHUNK = SEQ // NW
HALF = CHUNK // 2
LB = 512


def _sc_body(x_hbm, tab_hbm, pe_hbm, out_hbm,
             idx0, idx1, pe_v, buf0, buf1, sbuf0, sbuf1,
             gsem0, gsem1, ssem0, ssem1, isem0, isem1):
    wid = lax.axis_index("s") * NC + lax.axis_index("c")
    l0 = wid * CHUNK

    pltpu.sync_copy(pe_hbm.at[pl.ds(l0, CHUNK)], pe_v)

    idxs = (idx0, idx1)
    bufs = (buf0, buf1)
    sbufs = (sbuf0, sbuf1)
    gsems = (gsem0, gsem1)
    ssems = (ssem0, ssem1)
    isems = (isem0, isem1)

    for p in range(2):
        pltpu.sync_copy(x_hbm.at[0, pl.ds(l0 + p * HALF, HALF)], idxs[p])
        pltpu.async_copy(tab_hbm.at[idxs[p]], bufs[p], gsems[p])

    def step(b, carry):
        for p in range(2):
            off = p * HALF
            buf, sbuf = bufs[p], sbufs[p]
            pltpu.make_async_copy(tab_hbm.at[idxs[p]], buf, gsems[p]).wait()

            @pl.when(b < BSUB - 1)
            def _():
                pltpu.async_copy(
                    x_hbm.at[b + 1, pl.ds(l0 + off, HALF)], idxs[p], isems[p]
                )

            @pl.when(b > 0)
            def _():
                pltpu.make_async_copy(
                    sbuf, out_hbm.at[b - 1, pl.ds(l0 + off, HALF)], ssems[p]
                ).wait()

            def row_body(r, c2):
                for c in range(DIM // 16):
                    sl = pl.ds(c * 16, 16)
                    sbuf[r, sl] = buf[r, sl] + pe_v[r + off, sl]
                return c2

            lax.fori_loop(0, HALF, row_body, 0)
            pltpu.async_copy(sbuf, out_hbm.at[b, pl.ds(l0 + off, HALF)], ssems[p])

            @pl.when(b < BSUB - 1)
            def _():
                pltpu.make_async_copy(
                    x_hbm.at[b + 1, pl.ds(l0 + off, HALF)], idxs[p], isems[p]
                ).wait()
                pltpu.async_copy(tab_hbm.at[idxs[p]], buf, gsems[p])

        return carry

    lax.fori_loop(0, BSUB, step, 0)

    for p in range(2):
        off = p * HALF
        pltpu.make_async_copy(
            sbufs[p], out_hbm.at[BSUB - 1, pl.ds(l0 + off, HALF)], ssems[p]
        ).wait()


def _make_sc_run():
    mesh = plsc.VectorSubcoreMesh(
        core_axis_name="c", subcore_axis_name="s", num_cores=NC, num_subcores=NS
    )
    return pl.kernel(
        _sc_body,
        out_type=jax.ShapeDtypeStruct((BSUB, SEQ, DIM), jnp.float32),
        mesh=mesh,
        scratch_types=[
            pltpu.VMEM((HALF,), jnp.int32),
            pltpu.VMEM((HALF,), jnp.int32),
            pltpu.VMEM((CHUNK, DIM), jnp.float32),
            pltpu.VMEM((HALF, PAD), jnp.float32),
            pltpu.VMEM((HALF, PAD), jnp.float32),
            pltpu.VMEM((HALF, DIM), jnp.float32),
            pltpu.VMEM((HALF, DIM), jnp.float32),
            pltpu.SemaphoreType.DMA,
            pltpu.SemaphoreType.DMA,
            pltpu.SemaphoreType.DMA,
            pltpu.SemaphoreType.DMA,
            pltpu.SemaphoreType.DMA,
            pltpu.SemaphoreType.DMA,
        ],
    )


def _t_body(a_ref, o_ref):
    eye = jnp.eye(DIM, dtype=jnp.float32)
    a = a_ref[0]
    o_ref[0] = lax.dot_general(
        eye, a, (((1,), (1,)), ((), ())), preferred_element_type=jnp.float32
    )


def _tc_transpose(g):
    return pl.pallas_call(
        _t_body,
        out_shape=jax.ShapeDtypeStruct((BSUB, DIM, SEQ), jnp.float32),
        grid=(BSUB, SEQ // LB),
        in_specs=[pl.BlockSpec((1, LB, DIM), lambda b, s: (b, s, 0))],
        out_specs=pl.BlockSpec((1, DIM, LB), lambda b, s: (b, 0, s)),
    )(g)


@jax.jit
def kernel(x, embedding_table, positional_encoding):
    pe2d = positional_encoding.reshape(SEQ, DIM)
    xi = x.astype(jnp.int32)
    tab_pad = jnp.pad(embedding_table, ((0, 0), (0, PAD - DIM)))

    run = _make_sc_run()
    parts = []
    for i in range(NSPLIT):
        g = run(xi[i * BSUB:(i + 1) * BSUB], tab_pad, pe2d)
        parts.append(_tc_transpose(g))
    out_t = jnp.concatenate(parts, axis=0)
    return out_t.transpose(0, 2, 1)


# speedup vs baseline: 1.0271x; 1.0271x over previous
import functools

import jax
import jax.numpy as jnp
from jax import lax
from jax.experimental import pallas as pl
from jax.experimental.pallas import tpu as pltpu
from jax.experimental.pallas import tpu_sc as plsc

NC = 2
NS = 16
NW = NC * NS

BATCH = 64
NSPLIT = 1
BSUB = BATCH // NSPLIT
SEQ = 8192
DIM = 64
PAD = 128
CHUNK = SEQ // NW
HALF = CHUNK // 2
LB = 512


def _sc_body(x_hbm, tab_hbm, pe_hbm, out_hbm,
             idx0, idx1, pe_v, buf0, buf1, sbuf0, sbuf1,
             gsem0, gsem1, ssem0, ssem1, isem0, isem1):
    wid = lax.axis_index("s") * NC + lax.axis_index("c")
    l0 = wid * CHUNK

    pltpu.sync_copy(pe_hbm.at[pl.ds(l0, CHUNK)], pe_v)

    idxs = (idx0, idx1)
    bufs = (buf0, buf1)
    sbufs = (sbuf0, sbuf1)
    gsems = (gsem0, gsem1)
    ssems = (ssem0, ssem1)
    isems = (isem0, isem1)

    for p in range(2):
        pltpu.sync_copy(x_hbm.at[0, pl.ds(l0 + p * HALF, HALF)], idxs[p])
        pltpu.async_copy(tab_hbm.at[idxs[p]], bufs[p], gsems[p])

    def step(b, carry):
        for p in range(2):
            off = p * HALF
            buf, sbuf = bufs[p], sbufs[p]
            pltpu.make_async_copy(tab_hbm.at[idxs[p]], buf, gsems[p]).wait()

            @pl.when(b < BSUB - 1)
            def _():
                pltpu.async_copy(
                    x_hbm.at[b + 1, pl.ds(l0 + off, HALF)], idxs[p], isems[p]
                )

            @pl.when(b > 0)
            def _():
                pltpu.make_async_copy(
                    sbuf, out_hbm.at[b - 1, pl.ds(l0 + off, HALF)], ssems[p]
                ).wait()

            def row_body(r, c2):
                for c in range(DIM // 16):
                    sl = pl.ds(c * 16, 16)
                    sbuf[r, sl] = buf[r, sl] + pe_v[r + off, sl]
                return c2

            lax.fori_loop(0, HALF, row_body, 0)
            pltpu.async_copy(sbuf, out_hbm.at[b, pl.ds(l0 + off, HALF)], ssems[p])

            @pl.when(b < BSUB - 1)
            def _():
                pltpu.make_async_copy(
                    x_hbm.at[b + 1, pl.ds(l0 + off, HALF)], idxs[p], isems[p]
                ).wait()
                pltpu.async_copy(tab_hbm.at[idxs[p]], buf, gsems[p])

        return carry

    lax.fori_loop(0, BSUB, step, 0)

    for p in range(2):
        off = p * HALF
        pltpu.make_async_copy(
            sbufs[p], out_hbm.at[BSUB - 1, pl.ds(l0 + off, HALF)], ssems[p]
        ).wait()


def _make_sc_run():
    mesh = plsc.VectorSubcoreMesh(
        core_axis_name="c", subcore_axis_name="s", num_cores=NC, num_subcores=NS
    )
    return pl.kernel(
        _sc_body,
        out_type=jax.ShapeDtypeStruct((BSUB, SEQ, DIM), jnp.float32),
        mesh=mesh,
        scratch_types=[
            pltpu.VMEM((HALF,), jnp.int32),
            pltpu.VMEM((HALF,), jnp.int32),
            pltpu.VMEM((CHUNK, DIM), jnp.float32),
            pltpu.VMEM((HALF, PAD), jnp.float32),
            pltpu.VMEM((HALF, PAD), jnp.float32),
            pltpu.VMEM((HALF, DIM), jnp.float32),
            pltpu.VMEM((HALF, DIM), jnp.float32),
            pltpu.SemaphoreType.DMA,
            pltpu.SemaphoreType.DMA,
            pltpu.SemaphoreType.DMA,
            pltpu.SemaphoreType.DMA,
            pltpu.SemaphoreType.DMA,
            pltpu.SemaphoreType.DMA,
        ],
    )


def _t_body(a_ref, o_ref):
    o_ref[0] = a_ref[0].T


def _tc_transpose(g):
    return pl.pallas_call(
        _t_body,
        out_shape=jax.ShapeDtypeStruct((BSUB, DIM, SEQ), jnp.float32),
        grid=(BSUB, SEQ // LB),
        in_specs=[pl.BlockSpec((1, LB, DIM), lambda b, s: (b, s, 0))],
        out_specs=pl.BlockSpec((1, DIM, LB), lambda b, s: (b, 0, s)),
    )(g)


@jax.jit
def kernel(x, embedding_table, positional_encoding):
    pe2d = positional_encoding.reshape(SEQ, DIM)
    xi = x.astype(jnp.int32)
    tab_pad = jnp.pad(embedding_table, ((0, 0), (0, PAD - DIM)))

    run = _make_sc_run()
    parts = []
    for i in range(NSPLIT):
        g = run(xi[i * BSUB:(i + 1) * BSUB], tab_pad, pe2d)
        parts.append(_tc_transpose(g))
    out_t = jnp.concatenate(parts, axis=0)
    return out_t.transpose(0, 2, 1)


# speedup vs baseline: 2.1723x; 2.1149x over previous
import jax
import jax.numpy as jnp
from jax import lax
from jax.experimental import pallas as pl
from jax.experimental.pallas import tpu as pltpu
from jax.experimental.pallas import tpu_sc as plsc

NC = 2
NS = 16
NW = NC * NS

BATCH = 64
SEQ = 8192
DIM = 64
PAD = 128
CHUNK = SEQ // NW
NR = 4
SLOT = CHUNK // NR


def _sc_body(x_hbm, tab_hbm, pe_hbm, out_hbm,
             idx0, idx1, idx2, idx3, pe_v,
             buf0, buf1, buf2, buf3, sbuf0, sbuf1, sbuf2, sbuf3,
             gsem0, gsem1, gsem2, gsem3,
             ssem0, ssem1, ssem2, ssem3,
             isem0, isem1, isem2, isem3):
    wid = lax.axis_index("s") * NC + lax.axis_index("c")
    l0 = wid * CHUNK

    pltpu.sync_copy(pe_hbm.at[pl.ds(l0, CHUNK)], pe_v)

    idxs = (idx0, idx1, idx2, idx3)
    bufs = (buf0, buf1, buf2, buf3)
    sbufs = (sbuf0, sbuf1, sbuf2, sbuf3)
    gsems = (gsem0, gsem1, gsem2, gsem3)
    ssems = (ssem0, ssem1, ssem2, ssem3)
    isems = (isem0, isem1, isem2, isem3)

    for p in range(NR):
        pltpu.sync_copy(x_hbm.at[0, pl.ds(l0 + p * SLOT, SLOT)], idxs[p])
        pltpu.async_copy(tab_hbm.at[idxs[p]], bufs[p], gsems[p])

    def step(b, carry):
        for p in range(NR):
            off = p * SLOT
            buf, sbuf = bufs[p], sbufs[p]
            pltpu.make_async_copy(tab_hbm.at[idxs[p]], buf, gsems[p]).wait()

            @pl.when(b < BATCH - 1)
            def _():
                pltpu.async_copy(
                    x_hbm.at[b + 1, pl.ds(l0 + off, SLOT)], idxs[p], isems[p]
                )

            @pl.when(b > 0)
            def _():
                pltpu.make_async_copy(
                    sbuf, out_hbm.at[b - 1, pl.ds(l0 + off, SLOT)], ssems[p]
                ).wait()

            def row_body(r, c2):
                for c in range(DIM // 16):
                    sl = pl.ds(c * 16, 16)
                    sbuf[r, sl] = buf[r, sl] + pe_v[r + off, sl]
                return c2

            lax.fori_loop(0, SLOT, row_body, 0)
            pltpu.async_copy(sbuf, out_hbm.at[b, pl.ds(l0 + off, SLOT)], ssems[p])

            @pl.when(b < BATCH - 1)
            def _():
                pltpu.make_async_copy(
                    x_hbm.at[b + 1, pl.ds(l0 + off, SLOT)], idxs[p], isems[p]
                ).wait()
                pltpu.async_copy(tab_hbm.at[idxs[p]], buf, gsems[p])

        return carry

    lax.fori_loop(0, BATCH, step, 0)

    for p in range(NR):
        off = p * SLOT
        pltpu.make_async_copy(
            sbufs[p], out_hbm.at[BATCH - 1, pl.ds(l0 + off, SLOT)], ssems[p]
        ).wait()


@jax.jit
def kernel(x, embedding_table, positional_encoding):
    pe2d = positional_encoding.reshape(SEQ, DIM)
    xi = x.astype(jnp.int32)
    tab_pad = jnp.pad(embedding_table, ((0, 0), (0, PAD - DIM)))

    mesh = plsc.VectorSubcoreMesh(
        core_axis_name="c", subcore_axis_name="s", num_cores=NC, num_subcores=NS
    )
    run = pl.kernel(
        _sc_body,
        out_type=jax.ShapeDtypeStruct((BATCH, SEQ, DIM), jnp.float32),
        mesh=mesh,
        scratch_types=(
            [pltpu.VMEM((SLOT,), jnp.int32)] * NR
            + [pltpu.VMEM((CHUNK, DIM), jnp.float32)]
            + [pltpu.VMEM((SLOT, PAD), jnp.float32)] * NR
            + [pltpu.VMEM((SLOT, DIM), jnp.float32)] * NR
            + [pltpu.SemaphoreType.DMA] * (3 * NR)
        ),
    )
    return run(xi, tab_pad, pe2d)
